# Initial kernel scaffold; baseline (speedup 1.0000x reference)
import functools

import jax
import jax.numpy as jnp
import numpy as np
from jax.experimental import pallas as pl
from jax.scipy.special import logsumexp

B = 2
SEQ = 1536
PRED = 512
ENC_IN = 7
D_MODEL = 1024
D_FF = 2048
NH = 16
DH = D_MODEL // NH
EL = 2
C_OUT = 7
BUCKET = 32
NHASH = 4
L = SEQ + PRED
N_BUCKETS = L // BUCKET


def _sinusoid_table(n_pos, d):
    pe = np.zeros((n_pos, d), dtype=np.float32)
    pos = np.arange(n_pos, dtype=np.float32)[:, None]
    div = np.exp(np.arange(0, d, 2, dtype=np.float32) * -(np.log(10000.0) / d))
    pe[:, 0::2] = np.sin(pos * div)
    pe[:, 1::2] = np.cos(pos * div)
    return jnp.asarray(pe)

_MIN_TAB = _sinusoid_table(4, D_MODEL)
_HOUR_TAB = _sinusoid_table(24, D_MODEL)
_WD_TAB = _sinusoid_table(7, D_MODEL)
_DAY_TAB = _sinusoid_table(32, D_MODEL)
_MON_TAB = _sinusoid_table(13, D_MODEL)
_POS_TAB = _sinusoid_table(L, D_MODEL)


def _layer_norm(x, g, b):
    mu = jnp.mean(x, axis=-1, keepdims=True)
    var = jnp.mean((x - mu) ** 2, axis=-1, keepdims=True)
    return (x - mu) / jnp.sqrt(var + 1e-5) * g + b


def _token_embed(x, w):
    Lx = x.shape[1]
    xp = jnp.concatenate([x[:, -1:, :], x, x[:, :1, :]], axis=1)
    xs = jnp.stack([xp[:, k:k + Lx, :] for k in range(3)], axis=-1)
    return jnp.einsum('blck,dck->bld', xs, w)


def _lsh_attention(x, p, rot):
    Bsz, Lq, D = x.shape
    qk = x @ p['to_qk']
    v = x @ p['to_v']

    def split_heads(t):
        return t.reshape(Bsz, Lq, NH, DH).transpose(0, 2, 1, 3).reshape(Bsz * NH, Lq, DH)

    qk = split_heads(qk)
    v = split_heads(v)
    N = Bsz * NH
    rv = jnp.einsum('nld,dhr->nhlr', qk, rot)
    rv = jnp.concatenate([rv, -rv], axis=-1)
    buckets = jnp.argmax(rv, axis=-1)
    offsets = (jnp.arange(NHASH) * N_BUCKETS)[None, :, None]
    buckets = (buckets + offsets).reshape(N, NHASH * Lq)
    ticker = jnp.broadcast_to(jnp.arange(NHASH * Lq)[None, :], (N, NHASH * Lq))
    buckets_and_t = Lq * buckets + (ticker % Lq)
    sticker = jnp.argsort(buckets_and_t, axis=-1)
    undo_sort = jnp.argsort(sticker, axis=-1)
    st = sticker % Lq
    sqk = jnp.take_along_axis(qk, st[..., None], axis=1)
    sv = jnp.take_along_axis(v, st[..., None], axis=1)
    n_chunks = NHASH * N_BUCKETS
    bq = sqk.reshape(N, n_chunks, BUCKET, DH)
    knorm = jnp.maximum(jnp.linalg.norm(sqk, axis=-1, keepdims=True), 1e-12)
    bk = (sqk / knorm).reshape(N, n_chunks, BUCKET, DH)
    bv = sv.reshape(N, n_chunks, BUCKET, DH)
    bqt = st.reshape(N, n_chunks, BUCKET)

    def look_one_back(t):
        prev = jnp.roll(t, 1, axis=1)
        return jnp.concatenate([t, prev], axis=2)

    bk = look_one_back(bk)
    bv = look_one_back(bv)
    bkt = look_one_back(bqt)
    dots = jnp.einsum('ncid,ncjd->ncij', bq, bk) * (DH ** -0.5)
    self_mask = bqt[:, :, :, None] == bkt[:, :, None, :]
    dots = jnp.where(self_mask, -5e4, dots)
    dots_lse = logsumexp(dots, axis=-1, keepdims=True)
    probs = jnp.exp(dots - dots_lse)
    bo = jnp.einsum('ncij,ncjd->ncid', probs, bv)
    so = bo.reshape(N, NHASH * Lq, DH)
    slogits = dots_lse.reshape(N, NHASH * Lq)
    o = jnp.take_along_axis(so, undo_sort[..., None], axis=1)
    lgt = jnp.take_along_axis(slogits, undo_sort, axis=1)
    o = o.reshape(N, NHASH, Lq, DH)
    lgt = lgt.reshape(N, NHASH, Lq, 1)
    w = jnp.exp(lgt - logsumexp(lgt, axis=1, keepdims=True))
    out = jnp.sum(o * w, axis=1)
    out = out.reshape(Bsz, NH, Lq, DH).transpose(0, 2, 1, 3).reshape(Bsz, Lq, D)
    return out @ p['to_out'] + p['b_out']


def _encoder_layer(x, p, rot):
    new_x = _lsh_attention(x, p, rot)
    x = x + new_x
    x = _layer_norm(x, p['ln1_g'], p['ln1_b'])
    y = jax.nn.gelu(x @ p['ff1_w'] + p['ff1_b'], approximate=False)
    y = y @ p['ff2_w'] + p['ff2_b']
    return _layer_norm(x + y, p['ln2_g'], p['ln2_b'])


def _proj_kernel(x_ref, w_ref, b_ref, o_ref):
    o_ref[...] = x_ref[...] @ w_ref[...] + b_ref[...]


def _final_proj(x2d, w, b):
    wp = jnp.zeros((D_MODEL, 128), jnp.float32).at[:, :C_OUT].set(w)
    bp = jnp.zeros((128,), jnp.float32).at[:C_OUT].set(b)
    out = pl.pallas_call(
        _proj_kernel,
        out_shape=jax.ShapeDtypeStruct((x2d.shape[0], 128), jnp.float32),
        grid=(x2d.shape[0] // 512,),
        in_specs=[
            pl.BlockSpec((512, D_MODEL), lambda i: (i, 0)),
            pl.BlockSpec((D_MODEL, 128), lambda i: (0, 0)),
            pl.BlockSpec((128,), lambda i: (0,)),
        ],
        out_specs=pl.BlockSpec((512, 128), lambda i: (i, 0)),
    )(x2d, wp, bp)
    return out[:, :C_OUT]


def kernel(x_enc, x_mark_enc, x_dec, x_mark_dec, params):
    x = jnp.concatenate([x_enc, x_dec[:, -PRED:, :]], axis=1)
    xm = jnp.concatenate([x_mark_enc, x_mark_dec[:, -PRED:, :]], axis=1)
    temp = (_MON_TAB[xm[:, :, 0]] + _DAY_TAB[xm[:, :, 1]] + _WD_TAB[xm[:, :, 2]]
            + _HOUR_TAB[xm[:, :, 3]] + _MIN_TAB[xm[:, :, 4]])
    enc = _token_embed(x, params['conv_w']) + temp + _POS_TAB[None, :x.shape[1], :]
    for i in range(EL):
        enc = _encoder_layer(enc, params['layers'][i], params['rotations'][i])
    enc = _layer_norm(enc, params['norm_g'], params['norm_b'])
    dec = _final_proj(enc.reshape(B * L, D_MODEL), params['proj_w'], params['proj_b'])
    dec = dec.reshape(B, L, C_OUT)
    return dec[:, -PRED:, :]


# speedup vs baseline: 1.0091x; 1.0091x over previous
import functools

import jax
import jax.numpy as jnp
import numpy as np
from jax.experimental import pallas as pl
from jax.scipy.special import logsumexp

B = 2
SEQ = 1536
PRED = 512
ENC_IN = 7
D_MODEL = 1024
D_FF = 2048
NH = 16
DH = D_MODEL // NH
EL = 2
C_OUT = 7
BUCKET = 32
NHASH = 4
L = SEQ + PRED
N_BUCKETS = L // BUCKET


def _sinusoid_table(n_pos, d):
    pe = np.zeros((n_pos, d), dtype=np.float32)
    pos = np.arange(n_pos, dtype=np.float32)[:, None]
    div = np.exp(np.arange(0, d, 2, dtype=np.float32) * -(np.log(10000.0) / d))
    pe[:, 0::2] = np.sin(pos * div)
    pe[:, 1::2] = np.cos(pos * div)
    return jnp.asarray(pe)

_MIN_TAB = _sinusoid_table(4, D_MODEL)
_HOUR_TAB = _sinusoid_table(24, D_MODEL)
_WD_TAB = _sinusoid_table(7, D_MODEL)
_DAY_TAB = _sinusoid_table(32, D_MODEL)
_MON_TAB = _sinusoid_table(13, D_MODEL)
_POS_TAB = _sinusoid_table(L, D_MODEL)


def _layer_norm(x, g, b):
    mu = jnp.mean(x, axis=-1, keepdims=True)
    var = jnp.mean((x - mu) ** 2, axis=-1, keepdims=True)
    return (x - mu) / jnp.sqrt(var + 1e-5) * g + b


def _token_embed(x, w):
    Lx = x.shape[1]
    xp = jnp.concatenate([x[:, -1:, :], x, x[:, :1, :]], axis=1)
    xs = jnp.stack([xp[:, k:k + Lx, :] for k in range(3)], axis=-1)
    return jnp.einsum('blck,dck->bld', xs, w)


def _lsh_attention(x, p, rot):
    Bsz, Lq, D = x.shape
    qk = x @ p['to_qk']
    v = x @ p['to_v']

    def split_heads(t):
        return t.reshape(Bsz, Lq, NH, DH).transpose(0, 2, 1, 3).reshape(Bsz * NH, Lq, DH)

    qk = split_heads(qk)
    v = split_heads(v)
    N = Bsz * NH
    rv = jnp.einsum('nld,dhr->nhlr', qk, rot)
    rv = jnp.concatenate([rv, -rv], axis=-1)
    buckets = jnp.argmax(rv, axis=-1)
    offsets = (jnp.arange(NHASH) * N_BUCKETS)[None, :, None]
    buckets = (buckets + offsets).reshape(N, NHASH * Lq)
    ticker = jnp.broadcast_to(jnp.arange(NHASH * Lq)[None, :], (N, NHASH * Lq))
    buckets_and_t = Lq * buckets + (ticker % Lq)
    sticker = jnp.argsort(buckets_and_t, axis=-1)
    undo_sort = jnp.argsort(sticker, axis=-1)
    st = sticker % Lq
    sqk = jnp.take_along_axis(qk, st[..., None], axis=1)
    sv = jnp.take_along_axis(v, st[..., None], axis=1)
    n_chunks = NHASH * N_BUCKETS
    bq = sqk.reshape(N, n_chunks, BUCKET, DH)
    knorm = jnp.maximum(jnp.linalg.norm(sqk, axis=-1, keepdims=True), 1e-12)
    bk = (sqk / knorm).reshape(N, n_chunks, BUCKET, DH)
    bv = sv.reshape(N, n_chunks, BUCKET, DH)
    bqt = st.reshape(N, n_chunks, BUCKET)

    def look_one_back(t):
        prev = jnp.roll(t, 1, axis=1)
        return jnp.concatenate([t, prev], axis=2)

    bk = look_one_back(bk)
    bv = look_one_back(bv)
    bkt = look_one_back(bqt)
    dots = jnp.einsum('ncid,ncjd->ncij', bq, bk) * (DH ** -0.5)
    self_mask = bqt[:, :, :, None] == bkt[:, :, None, :]
    dots = jnp.where(self_mask, -5e4, dots)
    dots_lse = logsumexp(dots, axis=-1, keepdims=True)
    probs = jnp.exp(dots - dots_lse)
    bo = jnp.einsum('ncij,ncjd->ncid', probs, bv)
    so = bo.reshape(N, NHASH * Lq, DH)
    slogits = dots_lse.reshape(N, NHASH * Lq)
    o = jnp.take_along_axis(so, undo_sort[..., None], axis=1)
    lgt = jnp.take_along_axis(slogits, undo_sort, axis=1)
    o = o.reshape(N, NHASH, Lq, DH)
    lgt = lgt.reshape(N, NHASH, Lq, 1)
    w = jnp.exp(lgt - logsumexp(lgt, axis=1, keepdims=True))
    out = jnp.sum(o * w, axis=1)
    out = out.reshape(Bsz, NH, Lq, DH).transpose(0, 2, 1, 3).reshape(Bsz, Lq, D)
    return out @ p['to_out'] + p['b_out']


def _ffn_kernel(x_ref, w1_ref, b1_ref, w2_ref, b2_ref, o_ref):
    h = jax.lax.dot_general(x_ref[...], w1_ref[...], (((1,), (0,)), ((), ()))) + b1_ref[...]
    h = 0.5 * h * (1.0 + jax.lax.erf(h * 0.7071067811865476))
    o_ref[...] = jax.lax.dot_general(h, w2_ref[...], (((1,), (0,)), ((), ()))) + b2_ref[...]


def _ffn_pallas(x2d, w1, b1, w2, b2):
    M = x2d.shape[0]
    return pl.pallas_call(
        _ffn_kernel,
        out_shape=jax.ShapeDtypeStruct((M, D_MODEL), jnp.float32),
        grid=(M // 512,),
        in_specs=[
            pl.BlockSpec((512, D_MODEL), lambda i: (i, 0)),
            pl.BlockSpec((D_MODEL, D_FF), lambda i: (0, 0)),
            pl.BlockSpec((D_FF,), lambda i: (0,)),
            pl.BlockSpec((D_FF, D_MODEL), lambda i: (0, 0)),
            pl.BlockSpec((D_MODEL,), lambda i: (0,)),
        ],
        out_specs=pl.BlockSpec((512, D_MODEL), lambda i: (i, 0)),
    )(x2d, w1, b1, w2, b2)


def _encoder_layer(x, p, rot):
    new_x = _lsh_attention(x, p, rot)
    x = x + new_x
    x = _layer_norm(x, p['ln1_g'], p['ln1_b'])
    y = _ffn_pallas(x.reshape(B * L, D_MODEL), p['ff1_w'], p['ff1_b'],
                    p['ff2_w'], p['ff2_b']).reshape(B, L, D_MODEL)
    return _layer_norm(x + y, p['ln2_g'], p['ln2_b'])


def _proj_kernel(x_ref, w_ref, b_ref, o_ref):
    o_ref[...] = jax.lax.dot_general(
        x_ref[...], w_ref[...], (((1,), (0,)), ((), ())),
        precision=jax.lax.Precision.HIGHEST) + b_ref[...]


def _final_proj(x2d, w, b):
    wp = jnp.zeros((D_MODEL, 128), jnp.float32).at[:, :C_OUT].set(w)
    bp = jnp.zeros((128,), jnp.float32).at[:C_OUT].set(b)
    out = pl.pallas_call(
        _proj_kernel,
        out_shape=jax.ShapeDtypeStruct((x2d.shape[0], 128), jnp.float32),
        grid=(x2d.shape[0] // 512,),
        in_specs=[
            pl.BlockSpec((512, D_MODEL), lambda i: (i, 0)),
            pl.BlockSpec((D_MODEL, 128), lambda i: (0, 0)),
            pl.BlockSpec((128,), lambda i: (0,)),
        ],
        out_specs=pl.BlockSpec((512, 128), lambda i: (i, 0)),
    )(x2d, wp, bp)
    return out[:, :C_OUT]


def kernel(x_enc, x_mark_enc, x_dec, x_mark_dec, params):
    x = jnp.concatenate([x_enc, x_dec[:, -PRED:, :]], axis=1)
    xm = jnp.concatenate([x_mark_enc, x_mark_dec[:, -PRED:, :]], axis=1)
    temp = (_MON_TAB[xm[:, :, 0]] + _DAY_TAB[xm[:, :, 1]] + _WD_TAB[xm[:, :, 2]]
            + _HOUR_TAB[xm[:, :, 3]] + _MIN_TAB[xm[:, :, 4]])
    enc = _token_embed(x, params['conv_w']) + temp + _POS_TAB[None, :x.shape[1], :]
    for i in range(EL):
        enc = _encoder_layer(enc, params['layers'][i], params['rotations'][i])
    enc = _layer_norm(enc, params['norm_g'], params['norm_b'])
    dec = (enc @ params['proj_w'] + params['proj_b'])
    return dec[:, -PRED:, :]


# speedup vs baseline: 3.2804x; 3.2508x over previous
import functools

import jax
import jax.numpy as jnp
import numpy as np
from jax.experimental import pallas as pl
from jax.scipy.special import logsumexp

B = 2
SEQ = 1536
PRED = 512
ENC_IN = 7
D_MODEL = 1024
D_FF = 2048
NH = 16
DH = D_MODEL // NH
EL = 2
C_OUT = 7
BUCKET = 32
NHASH = 4
L = SEQ + PRED
N_BUCKETS = L // BUCKET
N = B * NH
BLQ = 512
NQB = L // BLQ
BLM = 512
SCALE = DH ** -0.5


def _sinusoid_table(n_pos, d):
    pe = np.zeros((n_pos, d), dtype=np.float32)
    pos = np.arange(n_pos, dtype=np.float32)[:, None]
    div = np.exp(np.arange(0, d, 2, dtype=np.float32) * -(np.log(10000.0) / d))
    pe[:, 0::2] = np.sin(pos * div)
    pe[:, 1::2] = np.cos(pos * div)
    return jnp.asarray(pe)

_MIN_TAB = _sinusoid_table(4, D_MODEL)
_HOUR_TAB = _sinusoid_table(24, D_MODEL)
_WD_TAB = _sinusoid_table(7, D_MODEL)
_DAY_TAB = _sinusoid_table(32, D_MODEL)
_MON_TAB = _sinusoid_table(13, D_MODEL)
_POS_TAB = _sinusoid_table(L, D_MODEL)


def _mm(a, b_mat):
    return jax.lax.dot_general(a, b_mat, (((1,), (0,)), ((), ())))


def _embed_kernel(xs_ref, w_ref, t_ref, o_ref):
    o_ref[...] = _mm(xs_ref[...], w_ref[...]) + t_ref[...]


def _embed(xs_pad, w_pad, temp):
    M = xs_pad.shape[0]
    return pl.pallas_call(
        _embed_kernel,
        out_shape=jax.ShapeDtypeStruct((M, D_MODEL), jnp.float32),
        grid=(M // BLM,),
        in_specs=[
            pl.BlockSpec((BLM, 24), lambda i: (i, 0)),
            pl.BlockSpec((24, D_MODEL), lambda i: (0, 0)),
            pl.BlockSpec((BLM, D_MODEL), lambda i: (i, 0)),
        ],
        out_specs=pl.BlockSpec((BLM, D_MODEL), lambda i: (i, 0)),
    )(xs_pad, w_pad, temp)


def _qkv_kernel(x_ref, wqk_ref, wv_ref, qk_ref, v_ref):
    x = x_ref[...]
    qk_ref[...] = _mm(x, wqk_ref[...])
    v_ref[...] = _mm(x, wv_ref[...])


def _qkv(x2d, wqk, wv):
    M = x2d.shape[0]
    return pl.pallas_call(
        _qkv_kernel,
        out_shape=[jax.ShapeDtypeStruct((M, D_MODEL), jnp.float32),
                   jax.ShapeDtypeStruct((M, D_MODEL), jnp.float32)],
        grid=(M // BLM,),
        in_specs=[
            pl.BlockSpec((BLM, D_MODEL), lambda i: (i, 0)),
            pl.BlockSpec((D_MODEL, D_MODEL), lambda i: (0, 0)),
            pl.BlockSpec((D_MODEL, D_MODEL), lambda i: (0, 0)),
        ],
        out_specs=[pl.BlockSpec((BLM, D_MODEL), lambda i: (i, 0)),
                   pl.BlockSpec((BLM, D_MODEL), lambda i: (i, 0))],
    )(x2d, wqk, wv)


def _attn_kernel(q_ref, k_ref, v_ref, cq_ref, ck_ref, o_ref, l_ref):
    qb = pl.program_id(2)
    q = q_ref[0]
    k = k_ref[0]
    v = v_ref[0]
    knorm = jnp.maximum(jnp.sqrt(jnp.sum(k * k, axis=1, keepdims=True)), 1e-12)
    khat = k / knorm
    dots = jax.lax.dot_general(q, khat, (((1,), (1,)), ((), ()))) * SCALE
    ci = cq_ref[0, 0]
    cj = ck_ref[0, 0]
    allow = (cj == ci) | (cj == ci - 1)
    iglob = qb * BLQ + jax.lax.broadcasted_iota(jnp.int32, (BLQ, L), 0)
    jidx = jax.lax.broadcasted_iota(jnp.int32, (BLQ, L), 1)
    diag = jidx == iglob
    dots = jnp.where(diag, -5e4, jnp.where(allow, dots, -1e9))
    m = jnp.max(dots, axis=1, keepdims=True)
    p = jnp.exp(dots - m)
    s = jnp.sum(p, axis=1, keepdims=True)
    l_ref[0, 0] = m + jnp.log(s)
    o_ref[0, 0] = _mm(p, v) / s


def _attn(qkh, vh, ch):
    ch_col = ch.reshape(N, NHASH, L, 1)
    ch_row = ch.reshape(N, NHASH, 1, L)
    o, lgt = pl.pallas_call(
        _attn_kernel,
        out_shape=[jax.ShapeDtypeStruct((N, NHASH, L, DH), jnp.float32),
                   jax.ShapeDtypeStruct((N, NHASH, L, 1), jnp.float32)],
        grid=(N, NHASH, NQB),
        in_specs=[
            pl.BlockSpec((1, BLQ, DH), lambda n, h, q: (n, q, 0)),
            pl.BlockSpec((1, L, DH), lambda n, h, q: (n, 0, 0)),
            pl.BlockSpec((1, L, DH), lambda n, h, q: (n, 0, 0)),
            pl.BlockSpec((1, 1, BLQ, 1), lambda n, h, q: (n, h, q, 0)),
            pl.BlockSpec((1, 1, 1, L), lambda n, h, q: (n, h, 0, 0)),
        ],
        out_specs=[pl.BlockSpec((1, 1, BLQ, DH), lambda n, h, q: (n, h, q, 0)),
                   pl.BlockSpec((1, 1, BLQ, 1), lambda n, h, q: (n, h, q, 0))],
    )(qkh, qkh, vh, ch_col, ch_row)
    return o, lgt.reshape(N, NHASH, L)


def _outln_kernel(o_ref, w_ref, b_ref, x_ref, g_ref, gb_ref, y_ref):
    t = _mm(o_ref[...], w_ref[...]) + b_ref[...] + x_ref[...]
    mu = jnp.mean(t, axis=1, keepdims=True)
    var = jnp.mean((t - mu) ** 2, axis=1, keepdims=True)
    y_ref[...] = (t - mu) / jnp.sqrt(var + 1e-5) * g_ref[...] + gb_ref[...]


def _outln(o2d, w, bias, x2d, g, gb):
    M = o2d.shape[0]
    return pl.pallas_call(
        _outln_kernel,
        out_shape=jax.ShapeDtypeStruct((M, D_MODEL), jnp.float32),
        grid=(M // BLM,),
        in_specs=[
            pl.BlockSpec((BLM, D_MODEL), lambda i: (i, 0)),
            pl.BlockSpec((D_MODEL, D_MODEL), lambda i: (0, 0)),
            pl.BlockSpec((D_MODEL,), lambda i: (0,)),
            pl.BlockSpec((BLM, D_MODEL), lambda i: (i, 0)),
            pl.BlockSpec((D_MODEL,), lambda i: (0,)),
            pl.BlockSpec((D_MODEL,), lambda i: (0,)),
        ],
        out_specs=pl.BlockSpec((BLM, D_MODEL), lambda i: (i, 0)),
    )(o2d, w, bias, x2d, g, gb)


def _ffn_kernel(x_ref, w1_ref, b1_ref, w2_ref, b2_ref, g_ref, gb_ref, y_ref):
    x = x_ref[...]
    h = _mm(x, w1_ref[...]) + b1_ref[...]
    h = 0.5 * h * (1.0 + jax.lax.erf(h * 0.7071067811865476))
    t = _mm(h, w2_ref[...]) + b2_ref[...] + x
    mu = jnp.mean(t, axis=1, keepdims=True)
    var = jnp.mean((t - mu) ** 2, axis=1, keepdims=True)
    y_ref[...] = (t - mu) / jnp.sqrt(var + 1e-5) * g_ref[...] + gb_ref[...]


def _ffn(x2d, w1, b1, w2, b2, g, gb):
    M = x2d.shape[0]
    return pl.pallas_call(
        _ffn_kernel,
        out_shape=jax.ShapeDtypeStruct((M, D_MODEL), jnp.float32),
        grid=(M // BLM,),
        in_specs=[
            pl.BlockSpec((BLM, D_MODEL), lambda i: (i, 0)),
            pl.BlockSpec((D_MODEL, D_FF), lambda i: (0, 0)),
            pl.BlockSpec((D_FF,), lambda i: (0,)),
            pl.BlockSpec((D_FF, D_MODEL), lambda i: (0, 0)),
            pl.BlockSpec((D_MODEL,), lambda i: (0,)),
            pl.BlockSpec((D_MODEL,), lambda i: (0,)),
            pl.BlockSpec((D_MODEL,), lambda i: (0,)),
        ],
        out_specs=pl.BlockSpec((BLM, D_MODEL), lambda i: (i, 0)),
    )(x2d, w1, b1, w2, b2, g, gb)


def _finproj_kernel(x_ref, g_ref, gb_ref, w_ref, b_ref, y_ref):
    x = x_ref[...]
    mu = jnp.mean(x, axis=1, keepdims=True)
    var = jnp.mean((x - mu) ** 2, axis=1, keepdims=True)
    t = (x - mu) / jnp.sqrt(var + 1e-5) * g_ref[...] + gb_ref[...]
    y_ref[...] = _mm(t, w_ref[...]) + b_ref[...]


def _finproj(x2d, g, gb, w, bias):
    M = x2d.shape[0]
    wp = jnp.zeros((D_MODEL, 128), jnp.float32).at[:, :C_OUT].set(w)
    bp = jnp.zeros((128,), jnp.float32).at[:C_OUT].set(bias)
    out = pl.pallas_call(
        _finproj_kernel,
        out_shape=jax.ShapeDtypeStruct((M, 128), jnp.float32),
        grid=(M // BLM,),
        in_specs=[
            pl.BlockSpec((BLM, D_MODEL), lambda i: (i, 0)),
            pl.BlockSpec((D_MODEL,), lambda i: (0,)),
            pl.BlockSpec((D_MODEL,), lambda i: (0,)),
            pl.BlockSpec((D_MODEL, 128), lambda i: (0, 0)),
            pl.BlockSpec((128,), lambda i: (0,)),
        ],
        out_specs=pl.BlockSpec((BLM, 128), lambda i: (i, 0)),
    )(x2d, g, gb, wp, bp)
    return out[:, :C_OUT]


def _lsh_layer(x2d, p, rot):
    qk2d, v2d = _qkv(x2d, p['to_qk'], p['to_v'])
    qkh = (qk2d.reshape(B, L, NH, DH).transpose(0, 2, 1, 3).reshape(N, L, DH))
    vh = (v2d.reshape(B, L, NH, DH).transpose(0, 2, 1, 3).reshape(N, L, DH))

    rv = jnp.einsum('nld,dhr->nhlr', qkh, rot)
    rv = jnp.concatenate([rv, -rv], axis=-1)
    buckets = jnp.argmax(rv, axis=-1).astype(jnp.int32)

    key = buckets * L + jnp.arange(L, dtype=jnp.int32)[None, None, :]
    sticker = jnp.argsort(key, axis=-1).astype(jnp.int32)
    rank = jnp.argsort(sticker, axis=-1).astype(jnp.int32)
    ch = rank // BUCKET

    o1, l1 = _attn(qkh, vh, ch)

    qb = sticker[:, :, :BUCKET]
    kb = jnp.roll(sticker[:, :, L - BUCKET:], 1, axis=1)

    def gather_rows(arr, idx):
        return jnp.take_along_axis(arr, idx.reshape(N, NHASH * BUCKET)[..., None],
                                   axis=1).reshape(N, NHASH, BUCKET, DH)

    qg = gather_rows(qkh, qb)
    kg = gather_rows(qkh, kb)
    vg = gather_rows(vh, kb)
    kg = kg / jnp.maximum(jnp.linalg.norm(kg, axis=-1, keepdims=True), 1e-12)
    s2 = jnp.einsum('nhid,nhjd->nhij', qg, kg) * SCALE
    s2 = jnp.where(qb[..., :, None] == kb[..., None, :], -5e4, s2)
    l2 = logsumexp(s2, axis=-1)
    o2 = jnp.einsum('nhij,nhjd->nhid', jnp.exp(s2 - l2[..., None]), vg)

    l1_at = jnp.take_along_axis(l1, qb, axis=2)
    o1_at = jnp.take_along_axis(o1, qb[..., None], axis=2)
    lm = jnp.logaddexp(l1_at, l2)
    om = (jnp.exp(l1_at - lm)[..., None] * o1_at
          + jnp.exp(l2 - lm)[..., None] * o2)
    ni = jnp.arange(N)[:, None, None]
    hi = jnp.arange(NHASH)[None, :, None]
    l1 = l1.at[ni, hi, qb].set(lm)
    o1 = o1.at[ni, hi, qb].set(om)

    w = jax.nn.softmax(l1, axis=1)
    out = jnp.einsum('nhl,nhld->nld', w, o1)
    out2d = (out.reshape(B, NH, L, DH).transpose(0, 2, 1, 3)
             .reshape(B * L, D_MODEL))

    x2d = _outln(out2d, p['to_out'], p['b_out'], x2d, p['ln1_g'], p['ln1_b'])
    return _ffn(x2d, p['ff1_w'], p['ff1_b'], p['ff2_w'], p['ff2_b'],
                p['ln2_g'], p['ln2_b'])


def kernel(x_enc, x_mark_enc, x_dec, x_mark_dec, params):
    x = jnp.concatenate([x_enc, x_dec[:, -PRED:, :]], axis=1)
    xm = jnp.concatenate([x_mark_enc, x_mark_dec[:, -PRED:, :]], axis=1)
    temp = (_MON_TAB[xm[:, :, 0]] + _DAY_TAB[xm[:, :, 1]] + _WD_TAB[xm[:, :, 2]]
            + _HOUR_TAB[xm[:, :, 3]] + _MIN_TAB[xm[:, :, 4]])
    temp = temp + _POS_TAB[None, :, :]
    xp = jnp.concatenate([x[:, -1:, :], x, x[:, :1, :]], axis=1)
    xs = jnp.stack([xp[:, k:k + L, :] for k in range(3)], axis=-1)
    xs = xs.reshape(B * L, ENC_IN * 3)
    xs_pad = jnp.zeros((B * L, 24), jnp.float32).at[:, :21].set(xs)
    w_conv = params['conv_w'].reshape(D_MODEL, 21).T
    w_pad = jnp.zeros((24, D_MODEL), jnp.float32).at[:21, :].set(w_conv)
    x2d = _embed(xs_pad, w_pad, temp.reshape(B * L, D_MODEL))
    for i in range(EL):
        x2d = _lsh_layer(x2d, params['layers'][i], params['rotations'][i])
    rows = x2d.reshape(B, L, D_MODEL)[:, -PRED:, :].reshape(B * PRED, D_MODEL)
    dec = _finproj(rows, params['norm_g'], params['norm_b'],
                   params['proj_w'], params['proj_b'])
    return dec.reshape(B, PRED, C_OUT)


# speedup vs baseline: 3.4363x; 1.0475x over previous
import functools

import jax
import jax.numpy as jnp
import numpy as np
from jax import lax
from jax.experimental import pallas as pl
from jax.experimental.pallas import tpu as pltpu
from jax.experimental.pallas import tpu_sc as plsc
from jax.scipy.special import logsumexp

B = 2
SEQ = 1536
PRED = 512
ENC_IN = 7
D_MODEL = 1024
D_FF = 2048
NH = 16
DH = D_MODEL // NH
EL = 2
C_OUT = 7
BUCKET = 32
NHASH = 4
L = SEQ + PRED
N_BUCKETS = L // BUCKET
N = B * NH
BLQ = 512
NQB = L // BLQ
BLM = 512
SCALE = DH ** -0.5


def _sinusoid_table(n_pos, d=D_MODEL):
    pos = jnp.arange(n_pos, dtype=jnp.float32)[:, None]
    div = jnp.exp(jnp.arange(0, d, 2, dtype=jnp.float32)
                  * jnp.float32(-(np.log(10000.0) / d)))
    ang = pos * div
    return jnp.stack([jnp.sin(ang), jnp.cos(ang)], axis=-1).reshape(n_pos, d)


def _mm(a, b_mat):
    return jax.lax.dot_general(a, b_mat, (((1,), (0,)), ((), ())))


def _embed_kernel(xs_ref, w_ref, t_ref, o_ref):
    o_ref[...] = _mm(xs_ref[...], w_ref[...]) + t_ref[...]


def _embed(xs_pad, w_pad, temp):
    M = xs_pad.shape[0]
    return pl.pallas_call(
        _embed_kernel,
        out_shape=jax.ShapeDtypeStruct((M, D_MODEL), jnp.float32),
        grid=(M // BLM,),
        in_specs=[
            pl.BlockSpec((BLM, 24), lambda i: (i, 0)),
            pl.BlockSpec((24, D_MODEL), lambda i: (0, 0)),
            pl.BlockSpec((BLM, D_MODEL), lambda i: (i, 0)),
        ],
        out_specs=pl.BlockSpec((BLM, D_MODEL), lambda i: (i, 0)),
    )(xs_pad, w_pad, temp)


def _qkv_kernel(x_ref, wqk_ref, wv_ref, qk_ref, v_ref):
    x = x_ref[...]
    qk_ref[...] = _mm(x, wqk_ref[...])
    v_ref[...] = _mm(x, wv_ref[...])


def _qkv(x2d, wqk, wv):
    M = x2d.shape[0]
    return pl.pallas_call(
        _qkv_kernel,
        out_shape=[jax.ShapeDtypeStruct((M, D_MODEL), jnp.float32),
                   jax.ShapeDtypeStruct((M, D_MODEL), jnp.float32)],
        grid=(M // BLM,),
        in_specs=[
            pl.BlockSpec((BLM, D_MODEL), lambda i: (i, 0)),
            pl.BlockSpec((D_MODEL, D_MODEL), lambda i: (0, 0)),
            pl.BlockSpec((D_MODEL, D_MODEL), lambda i: (0, 0)),
        ],
        out_specs=[pl.BlockSpec((BLM, D_MODEL), lambda i: (i, 0)),
                   pl.BlockSpec((BLM, D_MODEL), lambda i: (i, 0))],
    )(x2d, wqk, wv)


def _attn_kernel(q_ref, k_ref, v_ref, cq_ref, ck_ref, o_ref, l_ref):
    qb = pl.program_id(2)
    q = q_ref[0]
    k = k_ref[0]
    v = v_ref[0]
    knorm = jnp.maximum(jnp.sqrt(jnp.sum(k * k, axis=1, keepdims=True)), 1e-12)
    khat = k / knorm
    dots = jax.lax.dot_general(q, khat, (((1,), (1,)), ((), ()))) * SCALE
    ci = cq_ref[0, 0]
    cj = ck_ref[0, 0]
    allow = (cj == ci) | (cj == ci - 1)
    iglob = qb * BLQ + jax.lax.broadcasted_iota(jnp.int32, (BLQ, L), 0)
    jidx = jax.lax.broadcasted_iota(jnp.int32, (BLQ, L), 1)
    diag = jidx == iglob
    dots = jnp.where(diag, -5e4, jnp.where(allow, dots, -1e9))
    m = jnp.max(dots, axis=1, keepdims=True)
    p = jnp.exp(dots - m)
    s = jnp.sum(p, axis=1, keepdims=True)
    l_ref[0, 0] = m + jnp.log(s)
    o_ref[0, 0] = _mm(p, v) / s


def _attn(qkh, vh, ch):
    ch_col = ch.reshape(N, NHASH, L, 1)
    ch_row = ch.reshape(N, NHASH, 1, L)
    o, lgt = pl.pallas_call(
        _attn_kernel,
        out_shape=[jax.ShapeDtypeStruct((N, NHASH, L, DH), jnp.float32),
                   jax.ShapeDtypeStruct((N, NHASH, L, 1), jnp.float32)],
        grid=(N, NHASH, NQB),
        in_specs=[
            pl.BlockSpec((1, BLQ, DH), lambda n, h, q: (n, q, 0)),
            pl.BlockSpec((1, L, DH), lambda n, h, q: (n, 0, 0)),
            pl.BlockSpec((1, L, DH), lambda n, h, q: (n, 0, 0)),
            pl.BlockSpec((1, 1, BLQ, 1), lambda n, h, q: (n, h, q, 0)),
            pl.BlockSpec((1, 1, 1, L), lambda n, h, q: (n, h, 0, 0)),
        ],
        out_specs=[pl.BlockSpec((1, 1, BLQ, DH), lambda n, h, q: (n, h, q, 0)),
                   pl.BlockSpec((1, 1, BLQ, 1), lambda n, h, q: (n, h, q, 0))],
    )(qkh, qkh, vh, ch_col, ch_row)
    return o, lgt.reshape(N, NHASH, L)


def _outln_kernel(o_ref, w_ref, b_ref, x_ref, g_ref, gb_ref, y_ref):
    t = _mm(o_ref[...], w_ref[...]) + b_ref[...] + x_ref[...]
    mu = jnp.mean(t, axis=1, keepdims=True)
    var = jnp.mean((t - mu) ** 2, axis=1, keepdims=True)
    y_ref[...] = (t - mu) / jnp.sqrt(var + 1e-5) * g_ref[...] + gb_ref[...]


def _outln(o2d, w, bias, x2d, g, gb):
    M = o2d.shape[0]
    return pl.pallas_call(
        _outln_kernel,
        out_shape=jax.ShapeDtypeStruct((M, D_MODEL), jnp.float32),
        grid=(M // BLM,),
        in_specs=[
            pl.BlockSpec((BLM, D_MODEL), lambda i: (i, 0)),
            pl.BlockSpec((D_MODEL, D_MODEL), lambda i: (0, 0)),
            pl.BlockSpec((D_MODEL,), lambda i: (0,)),
            pl.BlockSpec((BLM, D_MODEL), lambda i: (i, 0)),
            pl.BlockSpec((D_MODEL,), lambda i: (0,)),
            pl.BlockSpec((D_MODEL,), lambda i: (0,)),
        ],
        out_specs=pl.BlockSpec((BLM, D_MODEL), lambda i: (i, 0)),
    )(o2d, w, bias, x2d, g, gb)


def _ffn_kernel(x_ref, w1_ref, b1_ref, w2_ref, b2_ref, g_ref, gb_ref, y_ref):
    x = x_ref[...]
    h = _mm(x, w1_ref[...]) + b1_ref[...]
    h = 0.5 * h * (1.0 + jax.lax.erf(h * 0.7071067811865476))
    t = _mm(h, w2_ref[...]) + b2_ref[...] + x
    mu = jnp.mean(t, axis=1, keepdims=True)
    var = jnp.mean((t - mu) ** 2, axis=1, keepdims=True)
    y_ref[...] = (t - mu) / jnp.sqrt(var + 1e-5) * g_ref[...] + gb_ref[...]


def _ffn(x2d, w1, b1, w2, b2, g, gb):
    M = x2d.shape[0]
    return pl.pallas_call(
        _ffn_kernel,
        out_shape=jax.ShapeDtypeStruct((M, D_MODEL), jnp.float32),
        grid=(M // BLM,),
        in_specs=[
            pl.BlockSpec((BLM, D_MODEL), lambda i: (i, 0)),
            pl.BlockSpec((D_MODEL, D_FF), lambda i: (0, 0)),
            pl.BlockSpec((D_FF,), lambda i: (0,)),
            pl.BlockSpec((D_FF, D_MODEL), lambda i: (0, 0)),
            pl.BlockSpec((D_MODEL,), lambda i: (0,)),
            pl.BlockSpec((D_MODEL,), lambda i: (0,)),
            pl.BlockSpec((D_MODEL,), lambda i: (0,)),
        ],
        out_specs=pl.BlockSpec((BLM, D_MODEL), lambda i: (i, 0)),
    )(x2d, w1, b1, w2, b2, g, gb)


def _finproj_kernel(x_ref, g_ref, gb_ref, w_ref, b_ref, y_ref):
    x = x_ref[...]
    mu = jnp.mean(x, axis=1, keepdims=True)
    var = jnp.mean((x - mu) ** 2, axis=1, keepdims=True)
    t = (x - mu) / jnp.sqrt(var + 1e-5) * g_ref[...] + gb_ref[...]
    y_ref[...] = _mm(t, w_ref[...]) + b_ref[...]


def _finproj(x2d, g, gb, w, bias):
    M = x2d.shape[0]
    wp = jnp.zeros((D_MODEL, 128), jnp.float32).at[:, :C_OUT].set(w)
    bp = jnp.zeros((128,), jnp.float32).at[:C_OUT].set(bias)
    out = pl.pallas_call(
        _finproj_kernel,
        out_shape=jax.ShapeDtypeStruct((M, 128), jnp.float32),
        grid=(M // BLM,),
        in_specs=[
            pl.BlockSpec((BLM, D_MODEL), lambda i: (i, 0)),
            pl.BlockSpec((D_MODEL,), lambda i: (0,)),
            pl.BlockSpec((D_MODEL,), lambda i: (0,)),
            pl.BlockSpec((D_MODEL, 128), lambda i: (0, 0)),
            pl.BlockSpec((128,), lambda i: (0,)),
        ],
        out_specs=pl.BlockSpec((BLM, 128), lambda i: (i, 0)),
    )(x2d, g, gb, wp, bp)
    return out[:, :C_OUT]


_NROWS = N * NHASH
_SEG = L // 16


def _rank_sc_body(bk_hbm, ch_hbm, qb_hbm, kb_hbm,
                  row_v, ch_v, qb_v, kb_v, cnt_v, off_v):
    nc = 2
    wid = lax.axis_index("s") * nc + lax.axis_index("c")
    lanes = lax.iota(jnp.int32, 16)
    ones16 = jnp.ones((16,), jnp.int32)
    for j in range(NHASH):
        r = wid * NHASH + j
        pltpu.sync_copy(bk_hbm.at[r], row_v)

        def zero_body(b, carry):
            cnt_v[b, :] = jnp.zeros((16,), jnp.int32)
            return carry
        lax.fori_loop(0, N_BUCKETS, zero_body, 0)

        def hist_body(i, carry):
            idx = lanes * _SEG + i
            bkt = plsc.load_gather(row_v, [idx])
            plsc.addupdate_scatter(cnt_v, [bkt, lanes], ones16)
            return carry
        lax.fori_loop(0, _SEG, hist_body, 0)

        def off_body(b, tot):
            crow = cnt_v[b, :]
            csum = plsc.cumsum(crow)
            off_v[b, :] = tot + csum - crow
            return tot + jnp.sum(crow)
        lax.fori_loop(0, N_BUCKETS, off_body, jnp.int32(0))

        def rank_body(i, carry):
            idx = lanes * _SEG + i
            bkt = plsc.load_gather(row_v, [idx])
            rk = plsc.load_gather(off_v, [bkt, lanes])
            plsc.store_scatter(off_v, [bkt, lanes], rk + 1)
            plsc.store_scatter(ch_v, [idx], lax.shift_right_logical(rk, 5))
            plsc.store_scatter(qb_v, [jnp.minimum(rk, 31)], idx,
                               mask=rk < BUCKET)
            plsc.store_scatter(kb_v, [jnp.maximum(rk - (L - BUCKET), 0)], idx,
                               mask=rk >= (L - BUCKET))
            return carry
        lax.fori_loop(0, _SEG, rank_body, 0)

        pltpu.sync_copy(ch_v, ch_hbm.at[r])
        pltpu.sync_copy(qb_v, qb_hbm.at[r])
        pltpu.sync_copy(kb_v, kb_hbm.at[r])


def _rank_sc(flat_buckets):
    mesh = plsc.VectorSubcoreMesh(core_axis_name="c", subcore_axis_name="s")
    f = pl.kernel(
        _rank_sc_body,
        mesh=mesh,
        compiler_params=pltpu.CompilerParams(needs_layout_passes=False),
        out_type=[jax.ShapeDtypeStruct((_NROWS, L), jnp.int32),
                  jax.ShapeDtypeStruct((_NROWS, BUCKET), jnp.int32),
                  jax.ShapeDtypeStruct((_NROWS, BUCKET), jnp.int32)],
        scratch_types=[pltpu.VMEM((L,), jnp.int32),
                       pltpu.VMEM((L,), jnp.int32),
                       pltpu.VMEM((BUCKET,), jnp.int32),
                       pltpu.VMEM((BUCKET,), jnp.int32),
                       pltpu.VMEM((N_BUCKETS, 16), jnp.int32),
                       pltpu.VMEM((N_BUCKETS, 16), jnp.int32)],
    )
    return f(flat_buckets)


def _lsh_layer(x2d, p, rot):
    qk2d, v2d = _qkv(x2d, p['to_qk'], p['to_v'])
    qkh = (qk2d.reshape(B, L, NH, DH).transpose(0, 2, 1, 3).reshape(N, L, DH))
    vh = (v2d.reshape(B, L, NH, DH).transpose(0, 2, 1, 3).reshape(N, L, DH))

    rv = jnp.einsum('nld,dhr->nhlr', qkh, rot)
    rv = jnp.concatenate([rv, -rv], axis=-1)
    buckets = jnp.argmax(rv, axis=-1).astype(jnp.int32)

    ch_flat, qb_flat, kb_flat = _rank_sc(buckets.reshape(_NROWS, L))
    ch = ch_flat.reshape(N, NHASH, L)

    o1, l1 = _attn(qkh, vh, ch)

    qb = qb_flat.reshape(N, NHASH, BUCKET)
    kb = jnp.roll(kb_flat.reshape(N, NHASH, BUCKET), 1, axis=1)

    def gather_rows(arr, idx):
        return jnp.take_along_axis(arr, idx.reshape(N, NHASH * BUCKET)[..., None],
                                   axis=1).reshape(N, NHASH, BUCKET, DH)

    qg = gather_rows(qkh, qb)
    kg = gather_rows(qkh, kb)
    vg = gather_rows(vh, kb)
    kg = kg / jnp.maximum(jnp.linalg.norm(kg, axis=-1, keepdims=True), 1e-12)
    s2 = jnp.einsum('nhid,nhjd->nhij', qg, kg) * SCALE
    s2 = jnp.where(qb[..., :, None] == kb[..., None, :], -5e4, s2)
    l2 = logsumexp(s2, axis=-1)
    o2 = jnp.einsum('nhij,nhjd->nhid', jnp.exp(s2 - l2[..., None]), vg)

    l1_at = jnp.take_along_axis(l1, qb, axis=2)
    o1_at = jnp.take_along_axis(o1, qb[..., None], axis=2)
    lm = jnp.logaddexp(l1_at, l2)
    om = (jnp.exp(l1_at - lm)[..., None] * o1_at
          + jnp.exp(l2 - lm)[..., None] * o2)
    ni = jnp.arange(N)[:, None, None]
    hi = jnp.arange(NHASH)[None, :, None]
    l1 = l1.at[ni, hi, qb].set(lm)
    o1 = o1.at[ni, hi, qb].set(om)

    w = jax.nn.softmax(l1, axis=1)
    out = jnp.einsum('nhl,nhld->nld', w, o1)
    out2d = (out.reshape(B, NH, L, DH).transpose(0, 2, 1, 3)
             .reshape(B * L, D_MODEL))

    x2d = _outln(out2d, p['to_out'], p['b_out'], x2d, p['ln1_g'], p['ln1_b'])
    return _ffn(x2d, p['ff1_w'], p['ff1_b'], p['ff2_w'], p['ff2_b'],
                p['ln2_g'], p['ln2_b'])


def kernel(x_enc, x_mark_enc, x_dec, x_mark_dec, params):
    x = jnp.concatenate([x_enc, x_dec[:, -PRED:, :]], axis=1)
    xm = jnp.concatenate([x_mark_enc, x_mark_dec[:, -PRED:, :]], axis=1)
    temp = (_sinusoid_table(13)[xm[:, :, 0]] + _sinusoid_table(32)[xm[:, :, 1]]
            + _sinusoid_table(7)[xm[:, :, 2]] + _sinusoid_table(24)[xm[:, :, 3]]
            + _sinusoid_table(4)[xm[:, :, 4]])
    temp = temp + _sinusoid_table(L)[None, :, :]
    xp = jnp.concatenate([x[:, -1:, :], x, x[:, :1, :]], axis=1)
    xs = jnp.stack([xp[:, k:k + L, :] for k in range(3)], axis=-1)
    xs = xs.reshape(B * L, ENC_IN * 3)
    xs_pad = jnp.zeros((B * L, 24), jnp.float32).at[:, :21].set(xs)
    w_conv = params['conv_w'].reshape(D_MODEL, 21).T
    w_pad = jnp.zeros((24, D_MODEL), jnp.float32).at[:21, :].set(w_conv)
    x2d = _embed(xs_pad, w_pad, temp.reshape(B * L, D_MODEL))
    for i in range(EL):
        x2d = _lsh_layer(x2d, params['layers'][i], params['rotations'][i])
    rows = x2d.reshape(B, L, D_MODEL)[:, -PRED:, :].reshape(B * PRED, D_MODEL)
    dec = _finproj(rows, params['norm_g'], params['norm_b'],
                   params['proj_w'], params['proj_b'])
    return dec.reshape(B, PRED, C_OUT)


# speedup vs baseline: 5.2227x; 1.5199x over previous
import functools

import jax
import jax.numpy as jnp
import numpy as np
from jax import lax
from jax.experimental import pallas as pl
from jax.experimental.pallas import tpu as pltpu
from jax.experimental.pallas import tpu_sc as plsc

B = 2
SEQ = 1536
PRED = 512
ENC_IN = 7
D_MODEL = 1024
D_FF = 2048
NH = 16
DH = D_MODEL // NH
EL = 2
C_OUT = 7
BUCKET = 32
NHASH = 4
L = SEQ + PRED
N_BUCKETS = L // BUCKET
N = B * NH
BLM = 512
SCALE = DH ** -0.5
BQ = 256
BAND = BQ + BUCKET
SOW = 128
_NROWS = N * NHASH
_SEG = L // 16
_GCH = 128


def _sinusoid_table(n_pos, d=D_MODEL):
    pos = jnp.arange(n_pos, dtype=jnp.float32)[:, None]
    div = jnp.exp(jnp.arange(0, d, 2, dtype=jnp.float32)
                  * jnp.float32(-(np.log(10000.0) / d)))
    ang = pos * div
    return jnp.stack([jnp.sin(ang), jnp.cos(ang)], axis=-1).reshape(n_pos, d)


def _mm(a, b_mat):
    return jax.lax.dot_general(a, b_mat, (((1,), (0,)), ((), ())))


def _embed_kernel(xs_ref, w_ref, t_ref, o_ref):
    o_ref[...] = _mm(xs_ref[...], w_ref[...]) + t_ref[...]


def _embed(xs_pad, w_pad, temp):
    M = xs_pad.shape[0]
    return pl.pallas_call(
        _embed_kernel,
        out_shape=jax.ShapeDtypeStruct((M, D_MODEL), jnp.float32),
        grid=(M // BLM,),
        in_specs=[
            pl.BlockSpec((BLM, 24), lambda i: (i, 0)),
            pl.BlockSpec((24, D_MODEL), lambda i: (0, 0)),
            pl.BlockSpec((BLM, D_MODEL), lambda i: (i, 0)),
        ],
        out_specs=pl.BlockSpec((BLM, D_MODEL), lambda i: (i, 0)),
    )(xs_pad, w_pad, temp)


def _qkv_kernel(x_ref, wqk_ref, wv_ref, qk_ref, v_ref):
    x = x_ref[...]
    qk_ref[...] = _mm(x, wqk_ref[...])
    v_ref[...] = _mm(x, wv_ref[...])


def _qkv(x2d, wqk, wv):
    M = x2d.shape[0]
    return pl.pallas_call(
        _qkv_kernel,
        out_shape=[jax.ShapeDtypeStruct((M, D_MODEL), jnp.float32),
                   jax.ShapeDtypeStruct((M, D_MODEL), jnp.float32)],
        grid=(M // BLM,),
        in_specs=[
            pl.BlockSpec((BLM, D_MODEL), lambda i: (i, 0)),
            pl.BlockSpec((D_MODEL, D_MODEL), lambda i: (0, 0)),
            pl.BlockSpec((D_MODEL, D_MODEL), lambda i: (0, 0)),
        ],
        out_specs=[pl.BlockSpec((BLM, D_MODEL), lambda i: (i, 0)),
                   pl.BlockSpec((BLM, D_MODEL), lambda i: (i, 0))],
    )(x2d, wqk, wv)


def _battn_kernel(q_ref, kx_ref, sq_ref, sx_ref, o_ref):
    qb = pl.program_id(2)
    start = qb * BQ
    q = q_ref[0, 0][:, :DH]
    kv = kx_ref[0, 0, pl.ds(start, BAND), :]
    kb = kv[:, :DH]
    vb = kv[:, DH:]
    stq = sq_ref[0, 0]
    stk = sx_ref[0, 0, :, pl.ds(start, BAND)]
    knorm = jnp.maximum(jnp.sqrt(jnp.sum(kb * kb, axis=1, keepdims=True)),
                        1e-12)
    dots = jax.lax.dot_general(q, kb / knorm,
                               (((1,), (1,)), ((), ()))) * SCALE
    iq = jax.lax.broadcasted_iota(jnp.int32, (BQ, BAND), 0) // BUCKET
    jk = jax.lax.broadcasted_iota(jnp.int32, (BQ, BAND), 1) // BUCKET
    allow = (jk == iq) | (jk == iq + 1)
    dots = jnp.where(allow, jnp.where(stq == stk, -5e4, dots), -1e9)
    m = jnp.max(dots, axis=1, keepdims=True)
    p = jnp.exp(dots - m)
    s = jnp.sum(p, axis=1, keepdims=True)
    o = _mm(p, vb) / s
    lse = m + jnp.log(s)
    o_ref[0, 0] = jnp.concatenate(
        [o, lse, jnp.zeros((BQ, SOW - DH - 1), jnp.float32)], axis=1)


def _battn(qvs4, kx, stq_col, sx_row):
    return pl.pallas_call(
        _battn_kernel,
        out_shape=jax.ShapeDtypeStruct((N, NHASH, L, SOW), jnp.float32),
        grid=(N, NHASH, L // BQ),
        in_specs=[
            pl.BlockSpec((1, 1, BQ, 2 * DH), lambda n, h, q: (n, h, q, 0)),
            pl.BlockSpec((1, 1, L + BUCKET, 2 * DH),
                         lambda n, h, q: (n, h, 0, 0)),
            pl.BlockSpec((1, 1, BQ, 1), lambda n, h, q: (n, h, q, 0)),
            pl.BlockSpec((1, 1, 1, L + BUCKET), lambda n, h, q: (n, h, 0, 0)),
        ],
        out_specs=pl.BlockSpec((1, 1, BQ, SOW), lambda n, h, q: (n, h, q, 0)),
    )(qvs4, kx, stq_col, sx_row)


def _outln_kernel(o_ref, w_ref, b_ref, x_ref, g_ref, gb_ref, y_ref):
    t = _mm(o_ref[...], w_ref[...]) + b_ref[...] + x_ref[...]
    mu = jnp.mean(t, axis=1, keepdims=True)
    var = jnp.mean((t - mu) ** 2, axis=1, keepdims=True)
    y_ref[...] = (t - mu) / jnp.sqrt(var + 1e-5) * g_ref[...] + gb_ref[...]


def _outln(o2d, w, bias, x2d, g, gb):
    M = o2d.shape[0]
    return pl.pallas_call(
        _outln_kernel,
        out_shape=jax.ShapeDtypeStruct((M, D_MODEL), jnp.float32),
        grid=(M // BLM,),
        in_specs=[
            pl.BlockSpec((BLM, D_MODEL), lambda i: (i, 0)),
            pl.BlockSpec((D_MODEL, D_MODEL), lambda i: (0, 0)),
            pl.BlockSpec((D_MODEL,), lambda i: (0,)),
            pl.BlockSpec((BLM, D_MODEL), lambda i: (i, 0)),
            pl.BlockSpec((D_MODEL,), lambda i: (0,)),
            pl.BlockSpec((D_MODEL,), lambda i: (0,)),
        ],
        out_specs=pl.BlockSpec((BLM, D_MODEL), lambda i: (i, 0)),
    )(o2d, w, bias, x2d, g, gb)


def _ffn_kernel(x_ref, w1_ref, b1_ref, w2_ref, b2_ref, g_ref, gb_ref, y_ref):
    x = x_ref[...]
    h = _mm(x, w1_ref[...]) + b1_ref[...]
    h = 0.5 * h * (1.0 + jax.lax.erf(h * 0.7071067811865476))
    t = _mm(h, w2_ref[...]) + b2_ref[...] + x
    mu = jnp.mean(t, axis=1, keepdims=True)
    var = jnp.mean((t - mu) ** 2, axis=1, keepdims=True)
    y_ref[...] = (t - mu) / jnp.sqrt(var + 1e-5) * g_ref[...] + gb_ref[...]


def _ffn(x2d, w1, b1, w2, b2, g, gb):
    M = x2d.shape[0]
    return pl.pallas_call(
        _ffn_kernel,
        out_shape=jax.ShapeDtypeStruct((M, D_MODEL), jnp.float32),
        grid=(M // BLM,),
        in_specs=[
            pl.BlockSpec((BLM, D_MODEL), lambda i: (i, 0)),
            pl.BlockSpec((D_MODEL, D_FF), lambda i: (0, 0)),
            pl.BlockSpec((D_FF,), lambda i: (0,)),
            pl.BlockSpec((D_FF, D_MODEL), lambda i: (0, 0)),
            pl.BlockSpec((D_MODEL,), lambda i: (0,)),
            pl.BlockSpec((D_MODEL,), lambda i: (0,)),
            pl.BlockSpec((D_MODEL,), lambda i: (0,)),
        ],
        out_specs=pl.BlockSpec((BLM, D_MODEL), lambda i: (i, 0)),
    )(x2d, w1, b1, w2, b2, g, gb)


def _finproj_kernel(x_ref, g_ref, gb_ref, w_ref, b_ref, y_ref):
    x = x_ref[...]
    mu = jnp.mean(x, axis=1, keepdims=True)
    var = jnp.mean((x - mu) ** 2, axis=1, keepdims=True)
    t = (x - mu) / jnp.sqrt(var + 1e-5) * g_ref[...] + gb_ref[...]
    y_ref[...] = _mm(t, w_ref[...]) + b_ref[...]


def _finproj(x2d, g, gb, w, bias):
    M = x2d.shape[0]
    wp = jnp.zeros((D_MODEL, 128), jnp.float32).at[:, :C_OUT].set(w)
    bp = jnp.zeros((128,), jnp.float32).at[:C_OUT].set(bias)
    out = pl.pallas_call(
        _finproj_kernel,
        out_shape=jax.ShapeDtypeStruct((M, 128), jnp.float32),
        grid=(M // BLM,),
        in_specs=[
            pl.BlockSpec((BLM, D_MODEL), lambda i: (i, 0)),
            pl.BlockSpec((D_MODEL,), lambda i: (0,)),
            pl.BlockSpec((D_MODEL,), lambda i: (0,)),
            pl.BlockSpec((D_MODEL, 128), lambda i: (0, 0)),
            pl.BlockSpec((128,), lambda i: (0,)),
        ],
        out_specs=pl.BlockSpec((BLM, 128), lambda i: (i, 0)),
    )(x2d, g, gb, wp, bp)
    return out[:, :C_OUT]


_SC_PARAMS = dict(
    compiler_params=pltpu.CompilerParams(needs_layout_passes=False))


def _rank_sc_body(bk_hbm, st_hbm, rk_hbm, row_v, st_v, rk_v, cnt_v, off_v):
    nc = 2
    wid = lax.axis_index("s") * nc + lax.axis_index("c")
    lanes = lax.iota(jnp.int32, 16)
    ones16 = jnp.ones((16,), jnp.int32)
    for j in range(NHASH):
        r = wid * NHASH + j
        qkbase = (r // NHASH) * L
        sobase = r * L
        pltpu.sync_copy(bk_hbm.at[r], row_v)

        def zero_body(b, carry):
            cnt_v[b, :] = jnp.zeros((16,), jnp.int32)
            return carry
        lax.fori_loop(0, N_BUCKETS, zero_body, 0)

        def hist_body(i, carry):
            idx = lanes * _SEG + i
            bkt = plsc.load_gather(row_v, [idx])
            plsc.addupdate_scatter(cnt_v, [bkt, lanes], ones16)
            return carry
        lax.fori_loop(0, _SEG, hist_body, 0)

        def off_body(b, tot):
            crow = cnt_v[b, :]
            csum = plsc.cumsum(crow)
            off_v[b, :] = tot + csum - crow
            return tot + jnp.sum(crow)
        lax.fori_loop(0, N_BUCKETS, off_body, jnp.int32(0))

        def rank_body(i, carry):
            idx = lanes * _SEG + i
            bkt = plsc.load_gather(row_v, [idx])
            rk = plsc.load_gather(off_v, [bkt, lanes])
            plsc.store_scatter(off_v, [bkt, lanes], rk + 1)
            plsc.store_scatter(rk_v, [idx], rk + sobase)
            plsc.store_scatter(st_v, [rk], idx + qkbase)
            return carry
        lax.fori_loop(0, _SEG, rank_body, 0)

        pltpu.sync_copy(st_v, st_hbm.at[r])
        pltpu.sync_copy(rk_v, rk_hbm.at[r])


def _rank_sc(flat_buckets):
    mesh = plsc.VectorSubcoreMesh(core_axis_name="c", subcore_axis_name="s")
    f = pl.kernel(
        _rank_sc_body,
        mesh=mesh,
        out_type=[jax.ShapeDtypeStruct((_NROWS, L), jnp.int32),
                  jax.ShapeDtypeStruct((_NROWS, L), jnp.int32)],
        scratch_types=[pltpu.VMEM((L,), jnp.int32),
                       pltpu.VMEM((L,), jnp.int32),
                       pltpu.VMEM((L,), jnp.int32),
                       pltpu.VMEM((N_BUCKETS, 16), jnp.int32),
                       pltpu.VMEM((N_BUCKETS, 16), jnp.int32)],
        **_SC_PARAMS,
    )
    return f(flat_buckets)


def _gather_sc_body(qv_hbm, st_hbm, qvs_hbm, idx_v, buf, sem1):
    nc = 2
    wid = lax.axis_index("s") * nc + lax.axis_index("c")
    for j in range(NHASH):
        r = wid * NHASH + j

        def chunk_body(c, carry):
            s = c * _GCH
            pltpu.sync_copy(st_hbm.at[r, pl.ds(s, _GCH)], idx_v)
            pltpu.async_copy(qv_hbm.at[idx_v], buf, sem1).wait()
            pltpu.sync_copy(buf, qvs_hbm.at[r, pl.ds(s, _GCH)])
            return carry
        lax.fori_loop(0, L // _GCH, chunk_body, 0)


def _gather_sc(qvflat, stg):
    mesh = plsc.VectorSubcoreMesh(core_axis_name="c", subcore_axis_name="s")
    f = pl.kernel(
        _gather_sc_body,
        mesh=mesh,
        out_type=jax.ShapeDtypeStruct((_NROWS, L, 2 * DH), jnp.float32),
        scratch_types=[pltpu.VMEM((_GCH,), jnp.int32),
                       pltpu.VMEM((_GCH, 2 * DH), jnp.float32),
                       pltpu.SemaphoreType.DMA],
        **_SC_PARAMS,
    )
    return f(qvflat, stg)


def _unsort_sc_body(so_hbm, rk_hbm, og_hbm, idx_v, obuf, sem1):
    nc = 2
    wid = lax.axis_index("s") * nc + lax.axis_index("c")
    for j in range(NHASH):
        r = wid * NHASH + j

        def chunk_body(c, carry):
            s = c * _GCH
            pltpu.sync_copy(rk_hbm.at[r, pl.ds(s, _GCH)], idx_v)
            pltpu.async_copy(so_hbm.at[idx_v], obuf, sem1).wait()
            pltpu.sync_copy(obuf, og_hbm.at[r, pl.ds(s, _GCH)])
            return carry
        lax.fori_loop(0, L // _GCH, chunk_body, 0)


def _unsort_sc(soflat, rkg):
    mesh = plsc.VectorSubcoreMesh(core_axis_name="c", subcore_axis_name="s")
    f = pl.kernel(
        _unsort_sc_body,
        mesh=mesh,
        out_type=jax.ShapeDtypeStruct((_NROWS, L, SOW), jnp.float32),
        scratch_types=[pltpu.VMEM((_GCH,), jnp.int32),
                       pltpu.VMEM((_GCH, SOW), jnp.float32),
                       pltpu.SemaphoreType.DMA],
        **_SC_PARAMS,
    )
    return f(soflat, rkg)


def _lsh_layer(x2d, p, rot):
    qk2d, v2d = _qkv(x2d, p['to_qk'], p['to_v'])
    qkh = (qk2d.reshape(B, L, NH, DH).transpose(0, 2, 1, 3).reshape(N, L, DH))
    vh = (v2d.reshape(B, L, NH, DH).transpose(0, 2, 1, 3).reshape(N, L, DH))

    rv = jnp.einsum('nld,dhr->nhlr', qkh, rot)
    rv = jnp.concatenate([rv, -rv], axis=-1)
    buckets = jnp.argmax(rv, axis=-1).astype(jnp.int32)

    stg, rkg = _rank_sc(buckets.reshape(_NROWS, L))

    qv = jnp.concatenate([qkh, vh], axis=-1)
    qvs = _gather_sc(qv.reshape(N * L, 2 * DH), stg)
    qvs4 = qvs.reshape(N, NHASH, L, 2 * DH)
    stg4 = stg.reshape(N, NHASH, L)

    kx = jnp.concatenate(
        [jnp.roll(qvs4[:, :, L - BUCKET:, :], 1, axis=1), qvs4], axis=2)
    sx = jnp.concatenate(
        [jnp.roll(stg4[:, :, L - BUCKET:], 1, axis=1), stg4], axis=2)

    so = _battn(qvs4, kx, stg4[..., None], sx[:, :, None, :])

    og = _unsort_sc(so.reshape(_NROWS * L, SOW), rkg)
    og4 = og.reshape(N, NHASH, L, SOW)
    o1 = og4[..., :DH]
    l1 = og4[..., DH]

    w = jax.nn.softmax(l1, axis=1)
    out = jnp.einsum('nhl,nhld->nld', w, o1)
    out2d = (out.reshape(B, NH, L, DH).transpose(0, 2, 1, 3)
             .reshape(B * L, D_MODEL))

    x2d = _outln(out2d, p['to_out'], p['b_out'], x2d, p['ln1_g'], p['ln1_b'])
    return _ffn(x2d, p['ff1_w'], p['ff1_b'], p['ff2_w'], p['ff2_b'],
                p['ln2_g'], p['ln2_b'])


def kernel(x_enc, x_mark_enc, x_dec, x_mark_dec, params):
    x = jnp.concatenate([x_enc, x_dec[:, -PRED:, :]], axis=1)
    xm = jnp.concatenate([x_mark_enc, x_mark_dec[:, -PRED:, :]], axis=1)
    temp = (_sinusoid_table(13)[xm[:, :, 0]] + _sinusoid_table(32)[xm[:, :, 1]]
            + _sinusoid_table(7)[xm[:, :, 2]] + _sinusoid_table(24)[xm[:, :, 3]]
            + _sinusoid_table(4)[xm[:, :, 4]])
    temp = temp + _sinusoid_table(L)[None, :, :]
    xp = jnp.concatenate([x[:, -1:, :], x, x[:, :1, :]], axis=1)
    xs = jnp.stack([xp[:, k:k + L, :] for k in range(3)], axis=-1)
    xs = xs.reshape(B * L, ENC_IN * 3)
    xs_pad = jnp.zeros((B * L, 24), jnp.float32).at[:, :21].set(xs)
    w_conv = params['conv_w'].reshape(D_MODEL, 21).T
    w_pad = jnp.zeros((24, D_MODEL), jnp.float32).at[:21, :].set(w_conv)
    x2d = _embed(xs_pad, w_pad, temp.reshape(B * L, D_MODEL))
    for i in range(EL):
        x2d = _lsh_layer(x2d, params['layers'][i], params['rotations'][i])
    rows = x2d.reshape(B, L, D_MODEL)[:, -PRED:, :].reshape(B * PRED, D_MODEL)
    dec = _finproj(rows, params['norm_g'], params['norm_b'],
                   params['proj_w'], params['proj_b'])
    return dec.reshape(B, PRED, C_OUT)
